# Initial kernel scaffold; baseline (speedup 1.0000x reference)
import jax
import jax.numpy as jnp
from jax.experimental import pallas as pl


def kernel(emb):
    raise NotImplementedError("write your pallas kernel here")


# speedup vs baseline: 2.6594x; 2.6594x over previous
import jax
import jax.numpy as jnp
from jax.experimental import pallas as pl
from jax.experimental.pallas import tpu as pltpu

_CLUSTERS = 300
_ITERS = 10
_CHUNK = 1024

_PREC = jax.lax.Precision.HIGHEST


def _dot(a, b, dims):
    return jax.lax.dot_general(a, b, (dims, ((), ())),
                               preferred_element_type=jnp.float32,
                               precision=_PREC)


def _kmeans_body(s_ref, m_ref, q_ref, b_ref, mo_ref,
                 samples_s, s2_s, means_s, sums_s, bins_s):
    it = pl.program_id(0)
    ch = pl.program_id(1)
    n_ch = pl.num_programs(1)
    rows = pl.ds(ch * _CHUNK, _CHUNK)
    col = jax.lax.broadcasted_iota(jnp.int32, (_CHUNK, _CLUSTERS), 1)

    @pl.when(it == 0)
    def _():
        chunk = s_ref[...]
        samples_s[rows, :] = chunk
        s2_s[rows, :] = jnp.sum(chunk * chunk, axis=1, keepdims=True)

        @pl.when(ch == 0)
        def _():
            means_s[...] = m_ref[...]

    @pl.when(it < _ITERS)
    def _():
        @pl.when(ch == 0)
        def _():
            sums_s[...] = jnp.zeros_like(sums_s)
            bins_s[...] = jnp.zeros_like(bins_s)

        samples = samples_s[rows, :]
        means = means_s[...]
        s2 = s2_s[rows, :]
        m2col = jnp.sum(means * means, axis=1, keepdims=True)
        m2row = jnp.transpose(m2col)
        dots = jax.lax.dot_general(samples, means, ((((1,), (1,))), ((), ())),
                                   preferred_element_type=jnp.float32,
                                   precision=jax.lax.Precision.DEFAULT)
        dists = -(s2 - 2.0 * dots + m2row)
        maxv = jnp.max(dists, axis=1, keepdims=True)
        cand = jnp.where(dists == maxv, col, _CLUSTERS)
        buckets = jnp.min(cand, axis=1, keepdims=True)
        oh = (col == buckets).astype(jnp.float32)

        b_ref[rows, :] = buckets
        ones = jnp.ones((_CHUNK, 1), jnp.float32)
        sums_s[...] += _dot(oh, samples, ((0,), (0,)))
        bins_s[...] += _dot(oh, ones, ((0,), (0,)))

        @pl.when(ch == n_ch - 1)
        def _():
            bins = bins_s[...]
            zero = bins == 0.0
            binsc = jnp.where(zero, 1.0, bins)
            new_means = sums_s[...] / binsc
            means_s[...] = jnp.where(zero, means_s[...], new_means)

    @pl.when(it == _ITERS)
    def _():
        buckets = b_ref[rows, :]
        oh = (col == buckets).astype(jnp.float32)
        q_ref[rows, :] = _dot(oh, means_s[...], ((1,), (0,)))

        @pl.when(ch == 0)
        def _():
            mo_ref[...] = means_s[...]


def kernel(emb):
    B, T, E = emb.shape
    n = B * T
    flat = emb.reshape(n, E)
    perm = jax.random.permutation(jax.random.key(42), n)[:_CLUSTERS]
    means0 = flat[perm]
    n_ch = n // _CHUNK

    quantized, buckets, means = pl.pallas_call(
        _kmeans_body,
        grid=(_ITERS + 1, n_ch),
        in_specs=[
            pl.BlockSpec((_CHUNK, E),
                         lambda it, ch: (jnp.where(it == 0, ch, 0), 0)),
            pl.BlockSpec((_CLUSTERS, E), lambda it, ch: (0, 0)),
        ],
        out_specs=[
            pl.BlockSpec((n, E), lambda it, ch: (0, 0)),
            pl.BlockSpec((n, 1), lambda it, ch: (0, 0)),
            pl.BlockSpec((_CLUSTERS, E), lambda it, ch: (0, 0)),
        ],
        out_shape=(
            jax.ShapeDtypeStruct((n, E), jnp.float32),
            jax.ShapeDtypeStruct((n, 1), jnp.int32),
            jax.ShapeDtypeStruct((_CLUSTERS, E), jnp.float32),
        ),
        scratch_shapes=[
            pltpu.VMEM((n, E), jnp.float32),
            pltpu.VMEM((n, 1), jnp.float32),
            pltpu.VMEM((_CLUSTERS, E), jnp.float32),
            pltpu.VMEM((_CLUSTERS, E), jnp.float32),
            pltpu.VMEM((_CLUSTERS, 1), jnp.float32),
        ],
    )(flat, means0)

    tokens = buckets.reshape(B, T)
    return quantized.reshape(B, T, E), tokens, means


# speedup vs baseline: 3.9233x; 1.4753x over previous
import jax
import jax.numpy as jnp
from jax.experimental import pallas as pl
from jax.experimental.pallas import tpu as pltpu

_CLUSTERS = 300
_ITERS = 10
_CHUNK = 1024


def _dot(a, b, dims):
    return jax.lax.dot_general(a, b, (dims, ((), ())),
                               preferred_element_type=jnp.float32,
                               precision=jax.lax.Precision.DEFAULT)


def _split3(x):
    hi = x.astype(jnp.bfloat16)
    r1 = x - hi.astype(jnp.float32)
    mid = r1.astype(jnp.bfloat16)
    lo = (r1 - mid.astype(jnp.float32)).astype(jnp.bfloat16)
    return hi, mid, lo


def _kmeans_body(s_ref, m_ref, q_ref, b_ref, mo_ref,
                 samples_s, hi_s, mid_s, lo_s, s2_s, means_s, sums_s, bins_s):
    it = pl.program_id(0)
    ch = pl.program_id(1)
    n_ch = pl.num_programs(1)
    rows = pl.ds(ch * _CHUNK, _CHUNK)
    col = jax.lax.broadcasted_iota(jnp.int32, (_CHUNK, _CLUSTERS), 1)

    @pl.when(it == 0)
    def _():
        chunk = s_ref[...]
        samples_s[rows, :] = chunk
        s2_s[rows, :] = jnp.sum(chunk * chunk, axis=1, keepdims=True)
        hi, mid, lo = _split3(chunk)
        hi_s[rows, :] = hi
        mid_s[rows, :] = mid
        lo_s[rows, :] = lo

        @pl.when(ch == 0)
        def _():
            means_s[...] = m_ref[...]

    @pl.when(it < _ITERS)
    def _():
        @pl.when(ch == 0)
        def _():
            sums_s[...] = jnp.zeros_like(sums_s)
            bins_s[...] = jnp.zeros_like(bins_s)

        samples = samples_s[rows, :]
        means = means_s[...]
        s2 = s2_s[rows, :]
        m2col = jnp.sum(means * means, axis=1, keepdims=True)
        m2row = jnp.transpose(m2col)
        dots = _dot(samples, means, ((1,), (1,)))
        dists = -(s2 - 2.0 * dots + m2row)
        maxv = jnp.max(dists, axis=1, keepdims=True)
        cand = jnp.where(dists == maxv, col, _CLUSTERS)
        buckets = jnp.min(cand, axis=1, keepdims=True)
        oh = (col == buckets).astype(jnp.bfloat16)

        b_ref[rows, :] = buckets
        ones = jnp.ones((_CHUNK, 1), jnp.bfloat16)
        sums_s[...] += ((_dot(oh, hi_s[rows, :], ((0,), (0,)))
                         + _dot(oh, mid_s[rows, :], ((0,), (0,))))
                        + _dot(oh, lo_s[rows, :], ((0,), (0,))))
        bins_s[...] += _dot(oh, ones, ((0,), (0,)))

        @pl.when(ch == n_ch - 1)
        def _():
            bins = bins_s[...]
            zero = bins == 0.0
            binsc = jnp.where(zero, 1.0, bins)
            new_means = sums_s[...] / binsc
            means_s[...] = jnp.where(zero, means_s[...], new_means)

    @pl.when(it == _ITERS)
    def _():
        buckets = b_ref[rows, :]
        oh = (col == buckets).astype(jnp.bfloat16)
        mhi, mmid, mlo = _split3(means_s[...])
        q_ref[rows, :] = ((_dot(oh, mhi, ((1,), (0,)))
                           + _dot(oh, mmid, ((1,), (0,))))
                          + _dot(oh, mlo, ((1,), (0,))))

        @pl.when(ch == 0)
        def _():
            mo_ref[...] = means_s[...]


def kernel(emb):
    B, T, E = emb.shape
    n = B * T
    flat = emb.reshape(n, E)
    perm = jax.random.permutation(jax.random.key(42), n)[:_CLUSTERS]
    means0 = flat[perm]
    n_ch = n // _CHUNK

    quantized, buckets, means = pl.pallas_call(
        _kmeans_body,
        grid=(_ITERS + 1, n_ch),
        in_specs=[
            pl.BlockSpec((_CHUNK, E),
                         lambda it, ch: (jnp.where(it == 0, ch, 0), 0)),
            pl.BlockSpec((_CLUSTERS, E), lambda it, ch: (0, 0)),
        ],
        out_specs=[
            pl.BlockSpec((n, E), lambda it, ch: (0, 0)),
            pl.BlockSpec((n, 1), lambda it, ch: (0, 0)),
            pl.BlockSpec((_CLUSTERS, E), lambda it, ch: (0, 0)),
        ],
        out_shape=(
            jax.ShapeDtypeStruct((n, E), jnp.float32),
            jax.ShapeDtypeStruct((n, 1), jnp.int32),
            jax.ShapeDtypeStruct((_CLUSTERS, E), jnp.float32),
        ),
        scratch_shapes=[
            pltpu.VMEM((n, E), jnp.float32),
            pltpu.VMEM((n, E), jnp.bfloat16),
            pltpu.VMEM((n, E), jnp.bfloat16),
            pltpu.VMEM((n, E), jnp.bfloat16),
            pltpu.VMEM((n, 1), jnp.float32),
            pltpu.VMEM((_CLUSTERS, E), jnp.float32),
            pltpu.VMEM((_CLUSTERS, E), jnp.float32),
            pltpu.VMEM((_CLUSTERS, 1), jnp.float32),
        ],
    )(flat, means0)

    tokens = buckets.reshape(B, T)
    return quantized.reshape(B, T, E), tokens, means


# speedup vs baseline: 5.5587x; 1.4168x over previous
import jax
import jax.numpy as jnp
from jax.experimental import pallas as pl
from jax.experimental.pallas import tpu as pltpu

_CLUSTERS = 300
_ITERS = 10
_CHUNK = 2048


def _dot(a, b, dims):
    return jax.lax.dot_general(a, b, (dims, ((), ())),
                               preferred_element_type=jnp.float32,
                               precision=jax.lax.Precision.DEFAULT)


def _split3(x):
    hi = x.astype(jnp.bfloat16)
    r1 = x - hi.astype(jnp.float32)
    mid = r1.astype(jnp.bfloat16)
    lo = (r1 - mid.astype(jnp.float32)).astype(jnp.bfloat16)
    return hi, mid, lo


def _kmeans_body(s_ref, m_ref, q_ref, b_ref, mo_ref,
                 s2x_s, sp_s, s2_s, means_s, m2_s, sums_s, bins_s):
    it = pl.program_id(0)
    ch = pl.program_id(1)
    n_ch = pl.num_programs(1)
    rows = pl.ds(ch * _CHUNK, _CHUNK)
    col = jax.lax.broadcasted_iota(jnp.int32, (_CHUNK, _CLUSTERS), 1)

    @pl.when(it == 0)
    def _():
        chunk = s_ref[...]
        s2x_s[rows, :] = chunk + chunk
        s2_s[rows, :] = jnp.sum(chunk * chunk, axis=1, keepdims=True)
        hi, mid, lo = _split3(chunk)
        sp_s[rows, pl.ds(0, 128)] = hi
        sp_s[rows, pl.ds(128, 128)] = mid
        sp_s[rows, pl.ds(256, 128)] = lo
        onecol = jax.lax.broadcasted_iota(jnp.int32, (_CHUNK, 128), 1) == 0
        sp_s[rows, pl.ds(384, 128)] = onecol.astype(jnp.bfloat16)

        @pl.when(ch == 0)
        def _():
            means_s[...] = m_ref[...]

    @pl.when(it < _ITERS)
    def _():
        @pl.when(ch == 0)
        def _():
            sums_s[...] = jnp.zeros_like(sums_s)
            bins_s[...] = jnp.zeros_like(bins_s)
            means = means_s[...]
            m2col = jnp.sum(means * means, axis=1, keepdims=True)
            m2_s[...] = jnp.transpose(m2col)

        dots2 = _dot(s2x_s[rows, :], means_s[...], ((1,), (1,)))
        nd = (s2_s[rows, :] - dots2) + m2_s[...]
        minv = jnp.min(nd, axis=1, keepdims=True)
        cand = jnp.where(nd == minv, col, _CLUSTERS)
        buckets = jnp.min(cand, axis=1, keepdims=True)
        oh = (col == buckets).astype(jnp.bfloat16)

        b_ref[rows, :] = buckets
        acc = _dot(oh, sp_s[rows, :], ((0,), (0,)))
        sums_s[...] += ((acc[:, 0:128] + acc[:, 128:256])
                        + acc[:, 256:384])
        bins_s[...] += acc[:, 384:385]

        @pl.when(ch == n_ch - 1)
        def _():
            bins = bins_s[...]
            zero = bins == 0.0
            binsc = jnp.where(zero, 1.0, bins)
            new_means = sums_s[...] / binsc
            means_s[...] = jnp.where(zero, means_s[...], new_means)

    @pl.when(it == _ITERS)
    def _():
        buckets = b_ref[rows, :]
        oh = (col == buckets).astype(jnp.bfloat16)
        mhi, mmid, mlo = _split3(means_s[...])
        q_ref[rows, :] = ((_dot(oh, mhi, ((1,), (0,)))
                           + _dot(oh, mmid, ((1,), (0,))))
                          + _dot(oh, mlo, ((1,), (0,))))

        @pl.when(ch == 0)
        def _():
            mo_ref[...] = means_s[...]


def kernel(emb):
    B, T, E = emb.shape
    n = B * T
    flat = emb.reshape(n, E)
    perm = jax.random.permutation(jax.random.key(42), n)[:_CLUSTERS]
    means0 = flat[perm]
    n_ch = n // _CHUNK

    quantized, buckets, means = pl.pallas_call(
        _kmeans_body,
        grid=(_ITERS + 1, n_ch),
        in_specs=[
            pl.BlockSpec((_CHUNK, E),
                         lambda it, ch: (jnp.where(it == 0, ch, 0), 0)),
            pl.BlockSpec((_CLUSTERS, E), lambda it, ch: (0, 0)),
        ],
        out_specs=[
            pl.BlockSpec((n, E), lambda it, ch: (0, 0)),
            pl.BlockSpec((n, 1), lambda it, ch: (0, 0)),
            pl.BlockSpec((_CLUSTERS, E), lambda it, ch: (0, 0)),
        ],
        out_shape=(
            jax.ShapeDtypeStruct((n, E), jnp.float32),
            jax.ShapeDtypeStruct((n, 1), jnp.int32),
            jax.ShapeDtypeStruct((_CLUSTERS, E), jnp.float32),
        ),
        scratch_shapes=[
            pltpu.VMEM((n, E), jnp.float32),
            pltpu.VMEM((n, 512), jnp.bfloat16),
            pltpu.VMEM((n, 1), jnp.float32),
            pltpu.VMEM((_CLUSTERS, E), jnp.float32),
            pltpu.VMEM((1, _CLUSTERS), jnp.float32),
            pltpu.VMEM((_CLUSTERS, E), jnp.float32),
            pltpu.VMEM((_CLUSTERS, 1), jnp.float32),
        ],
    )(flat, means0)

    tokens = buckets.reshape(B, T)
    return quantized.reshape(B, T, E), tokens, means
